# baseline (speedup 1.0000x reference)
import functools

import jax
import jax.numpy as jnp
from jax import lax
from jax.experimental import pallas as pl
from jax.experimental.pallas import tpu as pltpu
from jax.experimental.pallas import tpu_sc as plsc

N = 10000
E = 320000
D = 128
DH = 64
ACT_C = 1.679177

NC, NS = 2, 16
NW = NC * NS
EPW = E // NW
CH = 80
NCH = EPW // CH

EH = E // 2
EPWH = EH // NW
CHH = 40
NCHH = EPWH // CHH

NB = 2000
EB = 3200
EBR = EB // 128

_SC_MESH = plsc.VectorSubcoreMesh(core_axis_name="c", subcore_axis_name="s")
_SC_PARAMS = pltpu.CompilerParams(use_tc_tiling_on_sc=False)


def _tables_body(nf_ref, w1_ref, w2_ref, wm4_ref, wout_ref, g1_ref, g2_ref):
    v = wout_ref[:, 0] / (jnp.sqrt(128.0) * 40.0)
    wm4v = wm4_ref[...] * v[None, :] / jnp.sqrt(64.0)
    dn = (((1,), (1,)), ((), ()))
    scale = ACT_C / jnp.sqrt(128.0)
    m1 = lax.dot_general(w1_ref[...], wm4v, dn,
                         preferred_element_type=jnp.float32) * scale
    m2 = lax.dot_general(w2_ref[...], wm4v, dn,
                         preferred_element_type=jnp.float32) * scale
    nf = nf_ref[...]
    g1_ref[...] = jnp.dot(nf, m1, preferred_element_type=jnp.float32)
    g2_ref[...] = jnp.dot(nf, m2, preferred_element_type=jnp.float32)


_tables_call = pl.pallas_call(
    _tables_body,
    grid=(N // NB,),
    in_specs=[
        pl.BlockSpec((NB, D), lambda i: (i, 0)),
        pl.BlockSpec((D, D), lambda i: (0, 0)),
        pl.BlockSpec((D, D), lambda i: (0, 0)),
        pl.BlockSpec((DH, D), lambda i: (0, 0)),
        pl.BlockSpec((D, 1), lambda i: (0, 0)),
    ],
    out_specs=[
        pl.BlockSpec((NB, DH), lambda i: (i, 0)),
        pl.BlockSpec((NB, DH), lambda i: (i, 0)),
    ],
    out_shape=[jax.ShapeDtypeStruct((N, DH), jnp.float32)] * 2,
)


@functools.partial(
    pl.kernel,
    mesh=_SC_MESH,
    out_type=jax.ShapeDtypeStruct((EH, D), jnp.float32),
    scratch_types=[
        pltpu.VMEM((NCHH, CHH), jnp.int32),
        pltpu.VMEM((NCHH, CHH), jnp.int32),
        pltpu.VMEM((CHH, DH), jnp.float32),
        pltpu.VMEM((CHH, DH), jnp.float32),
        pltpu.VMEM((CHH, DH), jnp.float32),
        pltpu.VMEM((CHH, DH), jnp.float32),
        pltpu.SemaphoreType.DMA,
        pltpu.SemaphoreType.DMA,
    ],
    compiler_params=_SC_PARAMS,
)
def _gather_k(g1_hbm, g2_hbm, snd_hbm, rcv_hbm, bond_hbm,
              sidx, ridx, rows_sa, rows_ra, rows_sb, rows_rb, sema, semb):
    wid = lax.axis_index("s") * NC + lax.axis_index("c")
    pltpu.sync_copy(snd_hbm.at[wid], sidx)
    pltpu.sync_copy(rcv_hbm.at[wid], ridx)

    def start(j, rows_s, rows_r, sem):
        pltpu.make_async_copy(g1_hbm.at[sidx.at[j]], rows_s, sem).start()
        pltpu.make_async_copy(g2_hbm.at[ridx.at[j]], rows_r, sem).start()

    def drain_write(j, rows_s, rows_r, sem):
        base = wid * EPWH + j * CHH
        pltpu.make_async_copy(g1_hbm.at[sidx.at[j]], rows_s, sem).wait()
        pltpu.make_async_copy(g2_hbm.at[ridx.at[j]], rows_r, sem).wait()
        pltpu.sync_copy(rows_s, bond_hbm.at[pl.ds(base, CHH), pl.ds(0, DH)])
        pltpu.sync_copy(rows_r, bond_hbm.at[pl.ds(base, CHH), pl.ds(DH, DH)])

    start(0, rows_sa, rows_ra, sema)

    @pl.loop(0, NCHH - 1, step=2)
    def _(j):
        start(j + 1, rows_sb, rows_rb, semb)
        drain_write(j, rows_sa, rows_ra, sema)
        start(j + 2, rows_sa, rows_ra, sema)
        drain_write(j + 1, rows_sb, rows_rb, semb)

    drain_write(NCHH - 1, rows_sa, rows_ra, sema)


def _edge_body(ef_ref, wm1_ref, wm2_ref, wm3_ref, bond_ref, ea_ref,
               p1_ref):
    w1 = (wm1_ref[...] * 0.25).astype(jnp.bfloat16)
    w2 = (wm2_ref[...] * (ACT_C / 8.0)).astype(jnp.bfloat16)
    w3 = (wm3_ref[...] * (ACT_C / 8.0)).astype(jnp.bfloat16)
    x = ef_ref[...].astype(jnp.bfloat16)
    h = jax.nn.silu(jnp.dot(x, w1, preferred_element_type=jnp.float32))
    h = jax.nn.silu(jnp.dot(h.astype(jnp.bfloat16), w2,
                            preferred_element_type=jnp.float32))
    h = jax.nn.silu(jnp.dot(h.astype(jnp.bfloat16), w3,
                            preferred_element_type=jnp.float32))
    bond = bond_ref[...]
    prod = h * (bond[:, :DH] + bond[:, DH:])
    ssum = jnp.sum(prod, axis=1).reshape(1, EBR, 128)
    p1_ref[...] = ea_ref[...] * ssum


_edge_call = pl.pallas_call(
    _edge_body,
    grid=(EH // EB,),
    in_specs=[
        pl.BlockSpec((EB, 16), lambda i: (i, 0)),
        pl.BlockSpec((16, DH), lambda i: (0, 0)),
        pl.BlockSpec((DH, DH), lambda i: (0, 0)),
        pl.BlockSpec((DH, DH), lambda i: (0, 0)),
        pl.BlockSpec((EB, D), lambda i: (i, 0)),
        pl.BlockSpec((1, EBR, 128), lambda i: (i, 0, 0)),
    ],
    out_specs=pl.BlockSpec((1, EBR, 128), lambda i: (i, 0, 0)),
    out_shape=jax.ShapeDtypeStruct((EH // EB, EBR, 128), jnp.float32),
)


@functools.partial(
    pl.kernel,
    mesh=_SC_MESH,
    out_type=jax.ShapeDtypeStruct((NC, N), jnp.float32),
    scratch_types=[
        pltpu.VMEM((NCH, CH), jnp.float32),
        pltpu.VMEM((NCH, CH), jnp.float32),
        pltpu.VMEM((NCH, CH), jnp.int32),
        pltpu.VMEM((NCH, CH), jnp.int32),
        pltpu.VMEM((N,), jnp.float32),
        pltpu.VMEM_SHARED((N,), jnp.float32),
    ],
    compiler_params=_SC_PARAMS,
)
def _scatter_k(p_hbm, rcv_hbm, snd_hbm, out_hbm,
               pbuf, nbuf, ridx, sidx, zbuf, chg):
    c = lax.axis_index("c")
    s = lax.axis_index("s")
    wid = s * NC + c
    pltpu.sync_copy(p_hbm.at[wid], pbuf)
    pltpu.sync_copy(rcv_hbm.at[wid], ridx)
    pltpu.sync_copy(snd_hbm.at[wid], sidx)

    @pl.loop(0, NCH)
    def _(j):
        @pl.loop(0, CH, step=16)
        def _(k):
            nbuf[j, pl.ds(k, 16)] = -pbuf[j, pl.ds(k, 16)]

    @pl.when(s == 0)
    def _():
        @pl.loop(0, N, step=16)
        def _(i):
            zbuf[pl.ds(i, 16)] = jnp.zeros((16,), jnp.float32)
        pltpu.sync_copy(zbuf, chg)

    plsc.subcore_barrier()

    @pl.loop(0, NCH)
    def _(j):
        pltpu.sync_copy(pbuf.at[j], chg.at[ridx.at[j]], add=True)
        pltpu.sync_copy(nbuf.at[j], chg.at[sidx.at[j]], add=True)

    plsc.subcore_barrier()

    @pl.when(s == 0)
    def _():
        pltpu.sync_copy(chg, out_hbm.at[c])


def _chg_body(part_ref, out_ref):
    parts = part_ref[...]
    out_ref[...] = (parts[0, :] + parts[1, :])[:, None]


_chg_call = pl.pallas_call(
    _chg_body,
    grid=(1,),
    in_specs=[pl.BlockSpec((NC, N), lambda i: (0, 0))],
    out_specs=pl.BlockSpec((N, 1), lambda i: (0, 0)),
    out_shape=jax.ShapeDtypeStruct((N, 1), jnp.float32),
)


def kernel(node_attrs, node_feats, edge_attrs, edge_feats, edge_index,
           edge_vectors, batch, num_graphs, W1, W2, Wm1, Wm2, Wm3, Wm4,
           W_out, W_mp):
    snd = edge_index[0]
    rcv = edge_index[1]
    snd_a = snd[:EH].reshape(NW, NCHH, CHH)
    rcv_a = rcv[:EH].reshape(NW, NCHH, CHH)
    snd_b = snd[EH:].reshape(NW, NCHH, CHH)
    rcv_b = rcv[EH:].reshape(NW, NCHH, CHH)

    g1, g2 = _tables_call(node_feats, W1, W2, Wm4, W_out)
    bond_a = _gather_k(g1, g2, snd_a, rcv_a)
    bond_b = _gather_k(g1, g2, snd_b, rcv_b)
    ea = edge_attrs.reshape(2, EH // EB, EBR, 128)
    p1a = _edge_call(edge_feats[:EH], Wm1, Wm2, Wm3, bond_a, ea[0])
    p1b = _edge_call(edge_feats[EH:], Wm1, Wm2, Wm3, bond_b, ea[1])
    p = jnp.concatenate([p1a.reshape(EH), p1b.reshape(EH)])
    partials = _scatter_k(p.reshape(NW, NCH, CH), rcv.reshape(NW, NCH, CH),
                          snd.reshape(NW, NCH, CH))
    charges = _chg_call(partials)
    return charges, p.reshape(E, 1)


# speedup vs baseline: 1.0521x; 1.0521x over previous
import functools

import jax
import jax.numpy as jnp
from jax import lax
from jax.experimental import pallas as pl
from jax.experimental.pallas import tpu as pltpu
from jax.experimental.pallas import tpu_sc as plsc

N = 10000
E = 320000
D = 128
DH = 64
ACT_C = 1.679177

NC, NS = 2, 16
NW = NC * NS
EPW = E // NW
CH = 80
NCH = EPW // CH

NB = 2000
EB = 2560
EBR = EB // 128

_SC_MESH = plsc.VectorSubcoreMesh(core_axis_name="c", subcore_axis_name="s")
_SC_PARAMS = pltpu.CompilerParams(use_tc_tiling_on_sc=False)


def _tables_body(nf_ref, w1_ref, w2_ref, wm4_ref, wout_ref, g1_ref, g2_ref):
    v = wout_ref[:, 0] / (jnp.sqrt(128.0) * 40.0)
    wm4v = wm4_ref[...] * v[None, :] / jnp.sqrt(64.0)
    dn = (((1,), (1,)), ((), ()))
    scale = ACT_C / jnp.sqrt(128.0)
    m1 = lax.dot_general(w1_ref[...], wm4v, dn,
                         preferred_element_type=jnp.float32) * scale
    m2 = lax.dot_general(w2_ref[...], wm4v, dn,
                         preferred_element_type=jnp.float32) * scale
    nf = nf_ref[...]
    g1_ref[...] = jnp.dot(nf, m1, preferred_element_type=jnp.float32)
    g2_ref[...] = jnp.dot(nf, m2, preferred_element_type=jnp.float32)


_tables_call = pl.pallas_call(
    _tables_body,
    grid=(N // NB,),
    in_specs=[
        pl.BlockSpec((NB, D), lambda i: (i, 0)),
        pl.BlockSpec((D, D), lambda i: (0, 0)),
        pl.BlockSpec((D, D), lambda i: (0, 0)),
        pl.BlockSpec((DH, D), lambda i: (0, 0)),
        pl.BlockSpec((D, 1), lambda i: (0, 0)),
    ],
    out_specs=[
        pl.BlockSpec((NB, DH), lambda i: (i, 0)),
        pl.BlockSpec((NB, DH), lambda i: (i, 0)),
    ],
    out_shape=[jax.ShapeDtypeStruct((N, DH), jnp.float32)] * 2,
)


@functools.partial(
    pl.kernel,
    mesh=_SC_MESH,
    out_type=jax.ShapeDtypeStruct((E, D), jnp.float32),
    scratch_types=[
        pltpu.VMEM((NCH, CH), jnp.int32),
        pltpu.VMEM((NCH, CH), jnp.int32),
        pltpu.VMEM((CH, DH), jnp.float32),
        pltpu.VMEM((CH, DH), jnp.float32),
        pltpu.VMEM((CH, DH), jnp.float32),
        pltpu.VMEM((CH, DH), jnp.float32),
        pltpu.SemaphoreType.DMA,
        pltpu.SemaphoreType.DMA,
    ],
    compiler_params=_SC_PARAMS,
)
def _gather_k(g1_hbm, g2_hbm, snd_hbm, rcv_hbm, bond_hbm,
              sidx, ridx, rows_sa, rows_ra, rows_sb, rows_rb, sema, semb):
    wid = lax.axis_index("s") * NC + lax.axis_index("c")
    pltpu.sync_copy(snd_hbm.at[wid], sidx)
    pltpu.sync_copy(rcv_hbm.at[wid], ridx)

    def start(j, rows_s, rows_r, sem):
        pltpu.make_async_copy(g1_hbm.at[sidx.at[j]], rows_s, sem).start()
        pltpu.make_async_copy(g2_hbm.at[ridx.at[j]], rows_r, sem).start()

    def drain_write(j, rows_s, rows_r, sem):
        base = wid * EPW + j * CH
        pltpu.make_async_copy(g1_hbm.at[sidx.at[j]], rows_s, sem).wait()
        pltpu.make_async_copy(g2_hbm.at[ridx.at[j]], rows_r, sem).wait()
        pltpu.sync_copy(rows_s, bond_hbm.at[pl.ds(base, CH), pl.ds(0, DH)])
        pltpu.sync_copy(rows_r, bond_hbm.at[pl.ds(base, CH), pl.ds(DH, DH)])

    start(0, rows_sa, rows_ra, sema)

    @pl.loop(0, NCH - 1, step=2)
    def _(j):
        start(j + 1, rows_sb, rows_rb, semb)
        drain_write(j, rows_sa, rows_ra, sema)
        start(j + 2, rows_sa, rows_ra, sema)
        drain_write(j + 1, rows_sb, rows_rb, semb)

    drain_write(NCH - 1, rows_sa, rows_ra, sema)


def _edge_body(ef_ref, wm1_ref, wm2_ref, wm3_ref, bond_ref, ea_ref,
               p1_ref):
    w1 = (wm1_ref[...] * 0.25).astype(jnp.bfloat16)
    w2 = (wm2_ref[...] * (ACT_C / 8.0)).astype(jnp.bfloat16)
    w3 = (wm3_ref[...] * (ACT_C / 8.0)).astype(jnp.bfloat16)
    x = ef_ref[...].astype(jnp.bfloat16)
    h = jax.nn.silu(jnp.dot(x, w1, preferred_element_type=jnp.float32))
    h = jax.nn.silu(jnp.dot(h.astype(jnp.bfloat16), w2,
                            preferred_element_type=jnp.float32))
    h = jax.nn.silu(jnp.dot(h.astype(jnp.bfloat16), w3,
                            preferred_element_type=jnp.float32))
    bond = bond_ref[...]
    prod = h * (bond[:, :DH] + bond[:, DH:])
    ssum = jnp.sum(prod, axis=1).reshape(1, EBR, 128)
    p1_ref[...] = ea_ref[...] * ssum


_edge_call = pl.pallas_call(
    _edge_body,
    grid=(E // EB,),
    in_specs=[
        pl.BlockSpec((EB, 16), lambda i: (i, 0)),
        pl.BlockSpec((16, DH), lambda i: (0, 0)),
        pl.BlockSpec((DH, DH), lambda i: (0, 0)),
        pl.BlockSpec((DH, DH), lambda i: (0, 0)),
        pl.BlockSpec((EB, D), lambda i: (i, 0)),
        pl.BlockSpec((1, EBR, 128), lambda i: (i, 0, 0)),
    ],
    out_specs=pl.BlockSpec((1, EBR, 128), lambda i: (i, 0, 0)),
    out_shape=jax.ShapeDtypeStruct((E // EB, EBR, 128), jnp.float32),
)


@functools.partial(
    pl.kernel,
    mesh=_SC_MESH,
    out_type=jax.ShapeDtypeStruct((NC, N), jnp.float32),
    scratch_types=[
        pltpu.VMEM((NCH, CH), jnp.float32),
        pltpu.VMEM((NCH, CH), jnp.float32),
        pltpu.VMEM((NCH, CH), jnp.int32),
        pltpu.VMEM((NCH, CH), jnp.int32),
        pltpu.VMEM((N,), jnp.float32),
        pltpu.VMEM_SHARED((N,), jnp.float32),
    ],
    compiler_params=_SC_PARAMS,
)
def _scatter_k(p_hbm, rcv_hbm, snd_hbm, out_hbm,
               pbuf, nbuf, ridx, sidx, zbuf, chg):
    c = lax.axis_index("c")
    s = lax.axis_index("s")
    wid = s * NC + c
    pltpu.sync_copy(p_hbm.at[wid], pbuf)
    pltpu.sync_copy(rcv_hbm.at[wid], ridx)
    pltpu.sync_copy(snd_hbm.at[wid], sidx)

    @pl.loop(0, NCH)
    def _(j):
        @pl.loop(0, CH, step=16)
        def _(k):
            nbuf[j, pl.ds(k, 16)] = -pbuf[j, pl.ds(k, 16)]

    @pl.when(s == 0)
    def _():
        @pl.loop(0, N, step=16)
        def _(i):
            zbuf[pl.ds(i, 16)] = jnp.zeros((16,), jnp.float32)
        pltpu.sync_copy(zbuf, chg)

    plsc.subcore_barrier()

    @pl.loop(0, NCH)
    def _(j):
        pltpu.sync_copy(pbuf.at[j], chg.at[ridx.at[j]], add=True)
        pltpu.sync_copy(nbuf.at[j], chg.at[sidx.at[j]], add=True)

    plsc.subcore_barrier()

    @pl.when(s == 0)
    def _():
        pltpu.sync_copy(chg, out_hbm.at[c])


def _chg_body(part_ref, out_ref):
    parts = part_ref[...]
    out_ref[...] = (parts[0, :] + parts[1, :])[:, None]


_chg_call = pl.pallas_call(
    _chg_body,
    grid=(1,),
    in_specs=[pl.BlockSpec((NC, N), lambda i: (0, 0))],
    out_specs=pl.BlockSpec((N, 1), lambda i: (0, 0)),
    out_shape=jax.ShapeDtypeStruct((N, 1), jnp.float32),
)


def kernel(node_attrs, node_feats, edge_attrs, edge_feats, edge_index,
           edge_vectors, batch, num_graphs, W1, W2, Wm1, Wm2, Wm3, Wm4,
           W_out, W_mp):
    sender = edge_index[0].reshape(NW, NCH, CH)
    receiver = edge_index[1].reshape(NW, NCH, CH)

    g1, g2 = _tables_call(node_feats, W1, W2, Wm4, W_out)
    bond = _gather_k(g1, g2, sender, receiver)
    ea = edge_attrs.reshape(E // EB, EBR, 128)
    p1 = _edge_call(edge_feats, Wm1, Wm2, Wm3, bond, ea)
    partials = _scatter_k(p1.reshape(NW, NCH, CH), receiver, sender)
    charges = _chg_call(partials)
    return charges, p1.reshape(E, 1)


# speedup vs baseline: 1.0823x; 1.0286x over previous
import functools

import jax
import jax.numpy as jnp
from jax import lax
from jax.experimental import pallas as pl
from jax.experimental.pallas import tpu as pltpu
from jax.experimental.pallas import tpu_sc as plsc

N = 10000
E = 320000
D = 128
DH = 64
ACT_C = 1.679177

NC, NS = 2, 16
NW = NC * NS
EPW = E // NW
CH = 80
NCH = EPW // CH

NB = 2000
EB = 3200
EBR = EB // 128

_SC_MESH = plsc.VectorSubcoreMesh(core_axis_name="c", subcore_axis_name="s")
_SC_PARAMS = pltpu.CompilerParams(use_tc_tiling_on_sc=False)


def _tables_body(nf_ref, w1_ref, w2_ref, wm4_ref, wout_ref, g1_ref, g2_ref):
    v = wout_ref[:, 0] / (jnp.sqrt(128.0) * 40.0)
    wm4v = wm4_ref[...] * v[None, :] / jnp.sqrt(64.0)
    dn = (((1,), (1,)), ((), ()))
    scale = ACT_C / jnp.sqrt(128.0)
    m1 = lax.dot_general(w1_ref[...], wm4v, dn,
                         preferred_element_type=jnp.float32) * scale
    m2 = lax.dot_general(w2_ref[...], wm4v, dn,
                         preferred_element_type=jnp.float32) * scale
    nf = nf_ref[...]
    g1_ref[...] = jnp.dot(nf, m1, preferred_element_type=jnp.float32)
    g2_ref[...] = jnp.dot(nf, m2, preferred_element_type=jnp.float32)


_tables_call = pl.pallas_call(
    _tables_body,
    grid=(N // NB,),
    in_specs=[
        pl.BlockSpec((NB, D), lambda i: (i, 0)),
        pl.BlockSpec((D, D), lambda i: (0, 0)),
        pl.BlockSpec((D, D), lambda i: (0, 0)),
        pl.BlockSpec((DH, D), lambda i: (0, 0)),
        pl.BlockSpec((D, 1), lambda i: (0, 0)),
    ],
    out_specs=[
        pl.BlockSpec((NB, DH), lambda i: (i, 0)),
        pl.BlockSpec((NB, DH), lambda i: (i, 0)),
    ],
    out_shape=[jax.ShapeDtypeStruct((N, DH), jnp.float32)] * 2,
)


@functools.partial(
    pl.kernel,
    mesh=_SC_MESH,
    out_type=jax.ShapeDtypeStruct((E, D), jnp.float32),
    scratch_types=[
        pltpu.VMEM((NCH, CH), jnp.int32),
        pltpu.VMEM((NCH, CH), jnp.int32),
        pltpu.VMEM((CH, DH), jnp.float32),
        pltpu.VMEM((CH, DH), jnp.float32),
        pltpu.VMEM((CH, DH), jnp.float32),
        pltpu.VMEM((CH, DH), jnp.float32),
        pltpu.SemaphoreType.DMA,
        pltpu.SemaphoreType.DMA,
    ],
    compiler_params=_SC_PARAMS,
)
def _gather_k(g1_hbm, g2_hbm, snd_hbm, rcv_hbm, bond_hbm,
              sidx, ridx, rows_sa, rows_ra, rows_sb, rows_rb, sema, semb):
    wid = lax.axis_index("s") * NC + lax.axis_index("c")
    pltpu.sync_copy(snd_hbm.at[wid], sidx)
    pltpu.sync_copy(rcv_hbm.at[wid], ridx)

    def start(j, rows_s, rows_r, sem):
        pltpu.make_async_copy(g1_hbm.at[sidx.at[j]], rows_s, sem).start()
        pltpu.make_async_copy(g2_hbm.at[ridx.at[j]], rows_r, sem).start()

    def drain_write(j, rows_s, rows_r, sem):
        base = wid * EPW + j * CH
        pltpu.make_async_copy(g1_hbm.at[sidx.at[j]], rows_s, sem).wait()
        pltpu.make_async_copy(g2_hbm.at[ridx.at[j]], rows_r, sem).wait()
        pltpu.sync_copy(rows_s, bond_hbm.at[pl.ds(base, CH), pl.ds(0, DH)])
        pltpu.sync_copy(rows_r, bond_hbm.at[pl.ds(base, CH), pl.ds(DH, DH)])

    start(0, rows_sa, rows_ra, sema)

    @pl.loop(0, NCH - 1, step=2)
    def _(j):
        start(j + 1, rows_sb, rows_rb, semb)
        drain_write(j, rows_sa, rows_ra, sema)
        start(j + 2, rows_sa, rows_ra, sema)
        drain_write(j + 1, rows_sb, rows_rb, semb)

    drain_write(NCH - 1, rows_sa, rows_ra, sema)


def _edge_body(ef_ref, wm1_ref, wm2_ref, wm3_ref, bond_ref, ea_ref,
               p1_ref):
    w1 = (wm1_ref[...] * 0.25).astype(jnp.bfloat16)
    w2 = (wm2_ref[...] * (ACT_C / 8.0)).astype(jnp.bfloat16)
    w3 = (wm3_ref[...] * (ACT_C / 8.0)).astype(jnp.bfloat16)
    x = ef_ref[...].astype(jnp.bfloat16)
    h = jax.nn.silu(jnp.dot(x, w1, preferred_element_type=jnp.float32))
    h = jax.nn.silu(jnp.dot(h.astype(jnp.bfloat16), w2,
                            preferred_element_type=jnp.float32))
    h = jax.nn.silu(jnp.dot(h.astype(jnp.bfloat16), w3,
                            preferred_element_type=jnp.float32))
    bond = bond_ref[...]
    prod = h * (bond[:, :DH] + bond[:, DH:])
    ssum = jnp.sum(prod, axis=1).reshape(1, EBR, 128)
    p1_ref[...] = ea_ref[...] * ssum


_edge_call = pl.pallas_call(
    _edge_body,
    grid=(E // EB,),
    in_specs=[
        pl.BlockSpec((EB, 16), lambda i: (i, 0)),
        pl.BlockSpec((16, DH), lambda i: (0, 0)),
        pl.BlockSpec((DH, DH), lambda i: (0, 0)),
        pl.BlockSpec((DH, DH), lambda i: (0, 0)),
        pl.BlockSpec((EB, D), lambda i: (i, 0)),
        pl.BlockSpec((1, EBR, 128), lambda i: (i, 0, 0)),
    ],
    out_specs=pl.BlockSpec((1, EBR, 128), lambda i: (i, 0, 0)),
    out_shape=jax.ShapeDtypeStruct((E // EB, EBR, 128), jnp.float32),
)


@functools.partial(
    pl.kernel,
    mesh=_SC_MESH,
    out_type=jax.ShapeDtypeStruct((NC, N), jnp.float32),
    scratch_types=[
        pltpu.VMEM((NCH, CH), jnp.float32),
        pltpu.VMEM((NCH, CH), jnp.float32),
        pltpu.VMEM((NCH, CH), jnp.int32),
        pltpu.VMEM((NCH, CH), jnp.int32),
        pltpu.VMEM((N,), jnp.float32),
        pltpu.VMEM_SHARED((N,), jnp.float32),
    ],
    compiler_params=_SC_PARAMS,
)
def _scatter_k(p_hbm, rcv_hbm, snd_hbm, out_hbm,
               pbuf, nbuf, ridx, sidx, zbuf, chg):
    c = lax.axis_index("c")
    s = lax.axis_index("s")
    wid = s * NC + c
    pltpu.sync_copy(p_hbm.at[wid], pbuf)
    pltpu.sync_copy(rcv_hbm.at[wid], ridx)
    pltpu.sync_copy(snd_hbm.at[wid], sidx)

    @pl.loop(0, NCH)
    def _(j):
        @pl.loop(0, CH, step=16)
        def _(k):
            nbuf[j, pl.ds(k, 16)] = -pbuf[j, pl.ds(k, 16)]

    @pl.when(s == 0)
    def _():
        @pl.loop(0, N, step=16)
        def _(i):
            zbuf[pl.ds(i, 16)] = jnp.zeros((16,), jnp.float32)
        pltpu.sync_copy(zbuf, chg)

    plsc.subcore_barrier()

    @pl.loop(0, NCH)
    def _(j):
        pltpu.sync_copy(pbuf.at[j], chg.at[ridx.at[j]], add=True)
        pltpu.sync_copy(nbuf.at[j], chg.at[sidx.at[j]], add=True)

    plsc.subcore_barrier()

    @pl.when(s == 0)
    def _():
        pltpu.sync_copy(chg, out_hbm.at[c])


def _chg_body(part_ref, out_ref):
    parts = part_ref[...]
    out_ref[...] = (parts[0, :] + parts[1, :])[:, None]


_chg_call = pl.pallas_call(
    _chg_body,
    grid=(1,),
    in_specs=[pl.BlockSpec((NC, N), lambda i: (0, 0))],
    out_specs=pl.BlockSpec((N, 1), lambda i: (0, 0)),
    out_shape=jax.ShapeDtypeStruct((N, 1), jnp.float32),
)


def kernel(node_attrs, node_feats, edge_attrs, edge_feats, edge_index,
           edge_vectors, batch, num_graphs, W1, W2, Wm1, Wm2, Wm3, Wm4,
           W_out, W_mp):
    sender = edge_index[0].reshape(NW, NCH, CH)
    receiver = edge_index[1].reshape(NW, NCH, CH)

    g1, g2 = _tables_call(node_feats, W1, W2, Wm4, W_out)
    bond = _gather_k(g1, g2, sender, receiver)
    ea = edge_attrs.reshape(E // EB, EBR, 128)
    p1 = _edge_call(edge_feats, Wm1, Wm2, Wm3, bond, ea)
    partials = _scatter_k(p1.reshape(NW, NCH, CH), receiver, sender)
    charges = _chg_call(partials)
    return charges, p1.reshape(E, 1)


# speedup vs baseline: 1.1234x; 1.0380x over previous
import functools

import jax
import jax.numpy as jnp
from jax import lax
from jax.experimental import pallas as pl
from jax.experimental.pallas import tpu as pltpu
from jax.experimental.pallas import tpu_sc as plsc

N = 10000
E = 320000
D = 128
DH = 64
ACT_C = 1.679177

NC, NS = 2, 16
NW = NC * NS
EPW = E // NW
CH = 80
NCH = EPW // CH

NB = 2000
EB = 6400
EBR = EB // 128

_SC_MESH = plsc.VectorSubcoreMesh(core_axis_name="c", subcore_axis_name="s")
_SC_PARAMS = pltpu.CompilerParams(use_tc_tiling_on_sc=False)


def _tables_body(nf_ref, w1_ref, w2_ref, wm4_ref, wout_ref, g1_ref, g2_ref):
    v = wout_ref[:, 0] / (jnp.sqrt(128.0) * 40.0)
    wm4v = wm4_ref[...] * v[None, :] / jnp.sqrt(64.0)
    dn = (((1,), (1,)), ((), ()))
    scale = ACT_C / jnp.sqrt(128.0)
    m1 = lax.dot_general(w1_ref[...], wm4v, dn,
                         preferred_element_type=jnp.float32) * scale
    m2 = lax.dot_general(w2_ref[...], wm4v, dn,
                         preferred_element_type=jnp.float32) * scale
    nf = nf_ref[...]
    g1_ref[...] = jnp.dot(nf, m1, preferred_element_type=jnp.float32)
    g2_ref[...] = jnp.dot(nf, m2, preferred_element_type=jnp.float32)


_tables_call = pl.pallas_call(
    _tables_body,
    grid=(N // NB,),
    in_specs=[
        pl.BlockSpec((NB, D), lambda i: (i, 0)),
        pl.BlockSpec((D, D), lambda i: (0, 0)),
        pl.BlockSpec((D, D), lambda i: (0, 0)),
        pl.BlockSpec((DH, D), lambda i: (0, 0)),
        pl.BlockSpec((D, 1), lambda i: (0, 0)),
    ],
    out_specs=[
        pl.BlockSpec((NB, DH), lambda i: (i, 0)),
        pl.BlockSpec((NB, DH), lambda i: (i, 0)),
    ],
    out_shape=[jax.ShapeDtypeStruct((N, DH), jnp.float32)] * 2,
)


@functools.partial(
    pl.kernel,
    mesh=_SC_MESH,
    out_type=jax.ShapeDtypeStruct((E, D), jnp.float32),
    scratch_types=[
        pltpu.VMEM((NCH, CH), jnp.int32),
        pltpu.VMEM((NCH, CH), jnp.int32),
        pltpu.VMEM((CH, DH), jnp.float32),
        pltpu.VMEM((CH, DH), jnp.float32),
        pltpu.VMEM((CH, DH), jnp.float32),
        pltpu.VMEM((CH, DH), jnp.float32),
        pltpu.SemaphoreType.DMA,
        pltpu.SemaphoreType.DMA,
    ],
    compiler_params=_SC_PARAMS,
)
def _gather_k(g1_hbm, g2_hbm, snd_hbm, rcv_hbm, bond_hbm,
              sidx, ridx, rows_sa, rows_ra, rows_sb, rows_rb, sema, semb):
    wid = lax.axis_index("s") * NC + lax.axis_index("c")
    pltpu.sync_copy(snd_hbm.at[wid], sidx)
    pltpu.sync_copy(rcv_hbm.at[wid], ridx)

    def start(j, rows_s, rows_r, sem):
        pltpu.make_async_copy(g1_hbm.at[sidx.at[j]], rows_s, sem).start()
        pltpu.make_async_copy(g2_hbm.at[ridx.at[j]], rows_r, sem).start()

    def drain_write(j, rows_s, rows_r, sem):
        base = wid * EPW + j * CH
        pltpu.make_async_copy(g1_hbm.at[sidx.at[j]], rows_s, sem).wait()
        pltpu.make_async_copy(g2_hbm.at[ridx.at[j]], rows_r, sem).wait()
        pltpu.sync_copy(rows_s, bond_hbm.at[pl.ds(base, CH), pl.ds(0, DH)])
        pltpu.sync_copy(rows_r, bond_hbm.at[pl.ds(base, CH), pl.ds(DH, DH)])

    start(0, rows_sa, rows_ra, sema)

    @pl.loop(0, NCH - 1, step=2)
    def _(j):
        start(j + 1, rows_sb, rows_rb, semb)
        drain_write(j, rows_sa, rows_ra, sema)
        start(j + 2, rows_sa, rows_ra, sema)
        drain_write(j + 1, rows_sb, rows_rb, semb)

    drain_write(NCH - 1, rows_sa, rows_ra, sema)


def _edge_body(ef_ref, wm1_ref, wm2_ref, wm3_ref, bond_ref, ea_ref,
               p1_ref):
    w1 = (wm1_ref[...] * 0.25).astype(jnp.bfloat16)
    w2 = (wm2_ref[...] * (ACT_C / 8.0)).astype(jnp.bfloat16)
    w3 = (wm3_ref[...] * (ACT_C / 8.0)).astype(jnp.bfloat16)
    x = ef_ref[...].astype(jnp.bfloat16)
    h = jax.nn.silu(jnp.dot(x, w1, preferred_element_type=jnp.float32))
    h = jax.nn.silu(jnp.dot(h.astype(jnp.bfloat16), w2,
                            preferred_element_type=jnp.float32))
    h = jax.nn.silu(jnp.dot(h.astype(jnp.bfloat16), w3,
                            preferred_element_type=jnp.float32))
    bond = bond_ref[...]
    prod = h * (bond[:, :DH] + bond[:, DH:])
    ssum = jnp.sum(prod, axis=1).reshape(1, EBR, 128)
    p1_ref[...] = ea_ref[...] * ssum


_edge_call = pl.pallas_call(
    _edge_body,
    grid=(E // EB,),
    in_specs=[
        pl.BlockSpec((EB, 16), lambda i: (i, 0)),
        pl.BlockSpec((16, DH), lambda i: (0, 0)),
        pl.BlockSpec((DH, DH), lambda i: (0, 0)),
        pl.BlockSpec((DH, DH), lambda i: (0, 0)),
        pl.BlockSpec((EB, D), lambda i: (i, 0)),
        pl.BlockSpec((1, EBR, 128), lambda i: (i, 0, 0)),
    ],
    out_specs=pl.BlockSpec((1, EBR, 128), lambda i: (i, 0, 0)),
    out_shape=jax.ShapeDtypeStruct((E // EB, EBR, 128), jnp.float32),
)


@functools.partial(
    pl.kernel,
    mesh=_SC_MESH,
    out_type=jax.ShapeDtypeStruct((NC, N), jnp.float32),
    scratch_types=[
        pltpu.VMEM((NCH, CH), jnp.float32),
        pltpu.VMEM((NCH, CH), jnp.float32),
        pltpu.VMEM((NCH, CH), jnp.int32),
        pltpu.VMEM((NCH, CH), jnp.int32),
        pltpu.VMEM((N,), jnp.float32),
        pltpu.VMEM_SHARED((N,), jnp.float32),
    ],
    compiler_params=_SC_PARAMS,
)
def _scatter_k(p_hbm, rcv_hbm, snd_hbm, out_hbm,
               pbuf, nbuf, ridx, sidx, zbuf, chg):
    c = lax.axis_index("c")
    s = lax.axis_index("s")
    wid = s * NC + c
    pltpu.sync_copy(p_hbm.at[wid], pbuf)
    pltpu.sync_copy(rcv_hbm.at[wid], ridx)
    pltpu.sync_copy(snd_hbm.at[wid], sidx)

    @pl.loop(0, NCH)
    def _(j):
        @pl.loop(0, CH, step=16)
        def _(k):
            nbuf[j, pl.ds(k, 16)] = -pbuf[j, pl.ds(k, 16)]

    @pl.when(s == 0)
    def _():
        @pl.loop(0, N, step=16)
        def _(i):
            zbuf[pl.ds(i, 16)] = jnp.zeros((16,), jnp.float32)
        pltpu.sync_copy(zbuf, chg)

    plsc.subcore_barrier()

    @pl.loop(0, NCH)
    def _(j):
        pltpu.sync_copy(pbuf.at[j], chg.at[ridx.at[j]], add=True)
        pltpu.sync_copy(nbuf.at[j], chg.at[sidx.at[j]], add=True)

    plsc.subcore_barrier()

    @pl.when(s == 0)
    def _():
        pltpu.sync_copy(chg, out_hbm.at[c])


def _chg_body(part_ref, out_ref):
    parts = part_ref[...]
    out_ref[...] = (parts[0, :] + parts[1, :])[:, None]


_chg_call = pl.pallas_call(
    _chg_body,
    grid=(1,),
    in_specs=[pl.BlockSpec((NC, N), lambda i: (0, 0))],
    out_specs=pl.BlockSpec((N, 1), lambda i: (0, 0)),
    out_shape=jax.ShapeDtypeStruct((N, 1), jnp.float32),
)


def kernel(node_attrs, node_feats, edge_attrs, edge_feats, edge_index,
           edge_vectors, batch, num_graphs, W1, W2, Wm1, Wm2, Wm3, Wm4,
           W_out, W_mp):
    sender = edge_index[0].reshape(NW, NCH, CH)
    receiver = edge_index[1].reshape(NW, NCH, CH)

    g1, g2 = _tables_call(node_feats, W1, W2, Wm4, W_out)
    bond = _gather_k(g1, g2, sender, receiver)
    ea = edge_attrs.reshape(E // EB, EBR, 128)
    p1 = _edge_call(edge_feats, Wm1, Wm2, Wm3, bond, ea)
    partials = _scatter_k(p1.reshape(NW, NCH, CH), receiver, sender)
    charges = _chg_call(partials)
    return charges, p1.reshape(E, 1)


# speedup vs baseline: 1.1240x; 1.0005x over previous
import functools

import jax
import jax.numpy as jnp
from jax import lax
from jax.experimental import pallas as pl
from jax.experimental.pallas import tpu as pltpu
from jax.experimental.pallas import tpu_sc as plsc

N = 10000
E = 320000
D = 128
DH = 64
ACT_C = 1.679177

NC, NS = 2, 16
NW = NC * NS
EPW = E // NW
CH = 80
NCH = EPW // CH

NB = 2000
EB = 12800
EBR = EB // 128

_SC_MESH = plsc.VectorSubcoreMesh(core_axis_name="c", subcore_axis_name="s")
_SC_PARAMS = pltpu.CompilerParams(use_tc_tiling_on_sc=False)


def _tables_body(nf_ref, w1_ref, w2_ref, wm4_ref, wout_ref, g1_ref, g2_ref):
    v = wout_ref[:, 0] / (jnp.sqrt(128.0) * 40.0)
    wm4v = wm4_ref[...] * v[None, :] / jnp.sqrt(64.0)
    dn = (((1,), (1,)), ((), ()))
    scale = ACT_C / jnp.sqrt(128.0)
    m1 = lax.dot_general(w1_ref[...], wm4v, dn,
                         preferred_element_type=jnp.float32) * scale
    m2 = lax.dot_general(w2_ref[...], wm4v, dn,
                         preferred_element_type=jnp.float32) * scale
    nf = nf_ref[...]
    g1_ref[...] = jnp.dot(nf, m1, preferred_element_type=jnp.float32)
    g2_ref[...] = jnp.dot(nf, m2, preferred_element_type=jnp.float32)


_tables_call = pl.pallas_call(
    _tables_body,
    grid=(N // NB,),
    in_specs=[
        pl.BlockSpec((NB, D), lambda i: (i, 0)),
        pl.BlockSpec((D, D), lambda i: (0, 0)),
        pl.BlockSpec((D, D), lambda i: (0, 0)),
        pl.BlockSpec((DH, D), lambda i: (0, 0)),
        pl.BlockSpec((D, 1), lambda i: (0, 0)),
    ],
    out_specs=[
        pl.BlockSpec((NB, DH), lambda i: (i, 0)),
        pl.BlockSpec((NB, DH), lambda i: (i, 0)),
    ],
    out_shape=[jax.ShapeDtypeStruct((N, DH), jnp.float32)] * 2,
)


@functools.partial(
    pl.kernel,
    mesh=_SC_MESH,
    out_type=jax.ShapeDtypeStruct((E, D), jnp.float32),
    scratch_types=[
        pltpu.VMEM((NCH, CH), jnp.int32),
        pltpu.VMEM((NCH, CH), jnp.int32),
        pltpu.VMEM((CH, DH), jnp.float32),
        pltpu.VMEM((CH, DH), jnp.float32),
        pltpu.VMEM((CH, DH), jnp.float32),
        pltpu.VMEM((CH, DH), jnp.float32),
        pltpu.SemaphoreType.DMA,
        pltpu.SemaphoreType.DMA,
    ],
    compiler_params=_SC_PARAMS,
)
def _gather_k(g1_hbm, g2_hbm, snd_hbm, rcv_hbm, bond_hbm,
              sidx, ridx, rows_sa, rows_ra, rows_sb, rows_rb, sema, semb):
    wid = lax.axis_index("s") * NC + lax.axis_index("c")
    pltpu.sync_copy(snd_hbm.at[wid], sidx)
    pltpu.sync_copy(rcv_hbm.at[wid], ridx)

    def start(j, rows_s, rows_r, sem):
        pltpu.make_async_copy(g1_hbm.at[sidx.at[j]], rows_s, sem).start()
        pltpu.make_async_copy(g2_hbm.at[ridx.at[j]], rows_r, sem).start()

    def drain_write(j, rows_s, rows_r, sem):
        base = wid * EPW + j * CH
        pltpu.make_async_copy(g1_hbm.at[sidx.at[j]], rows_s, sem).wait()
        pltpu.make_async_copy(g2_hbm.at[ridx.at[j]], rows_r, sem).wait()
        pltpu.sync_copy(rows_s, bond_hbm.at[pl.ds(base, CH), pl.ds(0, DH)])
        pltpu.sync_copy(rows_r, bond_hbm.at[pl.ds(base, CH), pl.ds(DH, DH)])

    start(0, rows_sa, rows_ra, sema)

    @pl.loop(0, NCH - 1, step=2)
    def _(j):
        start(j + 1, rows_sb, rows_rb, semb)
        drain_write(j, rows_sa, rows_ra, sema)
        start(j + 2, rows_sa, rows_ra, sema)
        drain_write(j + 1, rows_sb, rows_rb, semb)

    drain_write(NCH - 1, rows_sa, rows_ra, sema)


def _edge_body(ef_ref, wm1_ref, wm2_ref, wm3_ref, bond_ref, ea_ref,
               p1_ref):
    w1 = (wm1_ref[...] * 0.25).astype(jnp.bfloat16)
    w2 = (wm2_ref[...] * (ACT_C / 8.0)).astype(jnp.bfloat16)
    w3 = (wm3_ref[...] * (ACT_C / 8.0)).astype(jnp.bfloat16)
    x = ef_ref[...].astype(jnp.bfloat16)
    h = jax.nn.silu(jnp.dot(x, w1, preferred_element_type=jnp.float32))
    h = jax.nn.silu(jnp.dot(h.astype(jnp.bfloat16), w2,
                            preferred_element_type=jnp.float32))
    h = jax.nn.silu(jnp.dot(h.astype(jnp.bfloat16), w3,
                            preferred_element_type=jnp.float32))
    bond = bond_ref[...]
    prod = h * (bond[:, :DH] + bond[:, DH:])
    ssum = jnp.sum(prod, axis=1).reshape(1, EBR, 128)
    p1_ref[...] = ea_ref[...] * ssum


_edge_call = pl.pallas_call(
    _edge_body,
    grid=(E // EB,),
    in_specs=[
        pl.BlockSpec((EB, 16), lambda i: (i, 0)),
        pl.BlockSpec((16, DH), lambda i: (0, 0)),
        pl.BlockSpec((DH, DH), lambda i: (0, 0)),
        pl.BlockSpec((DH, DH), lambda i: (0, 0)),
        pl.BlockSpec((EB, D), lambda i: (i, 0)),
        pl.BlockSpec((1, EBR, 128), lambda i: (i, 0, 0)),
    ],
    out_specs=pl.BlockSpec((1, EBR, 128), lambda i: (i, 0, 0)),
    out_shape=jax.ShapeDtypeStruct((E // EB, EBR, 128), jnp.float32),
)


@functools.partial(
    pl.kernel,
    mesh=_SC_MESH,
    out_type=jax.ShapeDtypeStruct((NC, N), jnp.float32),
    scratch_types=[
        pltpu.VMEM((NCH, CH), jnp.float32),
        pltpu.VMEM((NCH, CH), jnp.float32),
        pltpu.VMEM((NCH, CH), jnp.int32),
        pltpu.VMEM((NCH, CH), jnp.int32),
        pltpu.VMEM((N,), jnp.float32),
        pltpu.VMEM_SHARED((N,), jnp.float32),
    ],
    compiler_params=_SC_PARAMS,
)
def _scatter_k(p_hbm, rcv_hbm, snd_hbm, out_hbm,
               pbuf, nbuf, ridx, sidx, zbuf, chg):
    c = lax.axis_index("c")
    s = lax.axis_index("s")
    wid = s * NC + c
    pltpu.sync_copy(p_hbm.at[wid], pbuf)
    pltpu.sync_copy(rcv_hbm.at[wid], ridx)
    pltpu.sync_copy(snd_hbm.at[wid], sidx)

    @pl.loop(0, NCH)
    def _(j):
        @pl.loop(0, CH, step=16)
        def _(k):
            nbuf[j, pl.ds(k, 16)] = -pbuf[j, pl.ds(k, 16)]

    @pl.when(s == 0)
    def _():
        @pl.loop(0, N, step=16)
        def _(i):
            zbuf[pl.ds(i, 16)] = jnp.zeros((16,), jnp.float32)
        pltpu.sync_copy(zbuf, chg)

    plsc.subcore_barrier()

    @pl.loop(0, NCH)
    def _(j):
        pltpu.sync_copy(pbuf.at[j], chg.at[ridx.at[j]], add=True)
        pltpu.sync_copy(nbuf.at[j], chg.at[sidx.at[j]], add=True)

    plsc.subcore_barrier()

    @pl.when(s == 0)
    def _():
        pltpu.sync_copy(chg, out_hbm.at[c])


def _chg_body(part_ref, out_ref):
    parts = part_ref[...]
    out_ref[...] = (parts[0, :] + parts[1, :])[:, None]


_chg_call = pl.pallas_call(
    _chg_body,
    grid=(1,),
    in_specs=[pl.BlockSpec((NC, N), lambda i: (0, 0))],
    out_specs=pl.BlockSpec((N, 1), lambda i: (0, 0)),
    out_shape=jax.ShapeDtypeStruct((N, 1), jnp.float32),
)


def kernel(node_attrs, node_feats, edge_attrs, edge_feats, edge_index,
           edge_vectors, batch, num_graphs, W1, W2, Wm1, Wm2, Wm3, Wm4,
           W_out, W_mp):
    sender = edge_index[0].reshape(NW, NCH, CH)
    receiver = edge_index[1].reshape(NW, NCH, CH)

    g1, g2 = _tables_call(node_feats, W1, W2, Wm4, W_out)
    bond = _gather_k(g1, g2, sender, receiver)
    ea = edge_attrs.reshape(E // EB, EBR, 128)
    p1 = _edge_call(edge_feats, Wm1, Wm2, Wm3, bond, ea)
    partials = _scatter_k(p1.reshape(NW, NCH, CH), receiver, sender)
    charges = _chg_call(partials)
    return charges, p1.reshape(E, 1)
